# baseline (speedup 1.0000x reference)
import functools

import jax
import jax.numpy as jnp
from jax import lax
from jax.experimental import pallas as pl
from jax.experimental.pallas import tpu as pltpu
from jax.experimental.pallas import tpu_sc as plsc

V = 1000
D = 1000
B = 1024
T = 200
N = B * T
NC = 2
NS = 16
NW = NC * NS
PER_W = N // NW
B_PER_W = PER_W // T
CH = 40
CPT = T // CH
NCH = PER_W // CH


def _sc_gather(table, idx):
    mesh = plsc.VectorSubcoreMesh(core_axis_name="c", subcore_axis_name="s")

    @functools.partial(
        pl.kernel,
        mesh=mesh,
        out_type=jax.ShapeDtypeStruct((B, T, D), jnp.float32),
        scratch_types=[
            pltpu.VMEM((PER_W,), jnp.int32),
            pltpu.VMEM((2, CH, D), jnp.float32),
            pltpu.SemaphoreType.DMA,
        ],
        compiler_params=pltpu.CompilerParams(use_tc_tiling_on_sc=False),
    )
    def k(table_hbm, idx_hbm, out_hbm, idx_v, rows_v, gsem):
        wid = lax.axis_index("s") * NC + lax.axis_index("c")
        base = wid * PER_W
        b0 = wid * B_PER_W
        pltpu.sync_copy(idx_hbm.at[pl.ds(base, PER_W)], idx_v)

        def start_gather(g, b):
            off = pl.multiple_of(g * CH, 8)
            pltpu.async_copy(
                table_hbm.at[idx_v.at[pl.ds(off, CH)]], rows_v.at[b], gsem
            )

        def wait_gather(b):
            pltpu.make_async_copy(
                table_hbm.at[pl.ds(0, CH)], rows_v.at[b], gsem
            ).wait()

        def store(g, b):
            pltpu.sync_copy(
                rows_v.at[b],
                out_hbm.at[b0 + g // CPT, pl.ds((g % CPT) * CH, CH)],
            )

        start_gather(0, 0)

        def body(i, carry):
            g0 = 2 * i
            start_gather(g0 + 1, 1)
            wait_gather(0)
            store(g0, 0)

            @pl.when(g0 + 2 < NCH)
            def _():
                start_gather(g0 + 2, 0)

            wait_gather(1)
            store(g0 + 1, 1)
            return carry

        lax.fori_loop(0, NCH // 2, body, 0)

    return k(table, idx)


def kernel(encoding, table):
    idx = encoding.reshape(-1).astype(jnp.int32)
    return _sc_gather(table, idx)
